# initial kernel scaffold (unmeasured)
import jax
import jax.numpy as jnp
from jax import lax
from jax.experimental import pallas as pl
from jax.experimental.pallas import tpu as pltpu

N_DEV = 8
HALF = 512


def kernel(x, w_mat):
    m_per, k = x.shape
    _, n_per = w_mat.shape
    assert m_per == 2 * HALF

    def body(x_ref, w_ref, out_ref, comm_cw, comm_ccw, stage,
             cw_send, cw_recv, ccw_send, ccw_recv, copy_sem):
        my = lax.axis_index("i")
        left = lax.rem(my - 1 + N_DEV, N_DEV)
        right = lax.rem(my + 1, N_DEV)

        barrier_sem = pltpu.get_barrier_semaphore()
        for nbr in (left, right):
            pl.semaphore_signal(
                barrier_sem, inc=1,
                device_id=(nbr,), device_id_type=pl.DeviceIdType.MESH,
            )
        pl.semaphore_wait(barrier_sem, 2)

        def gemm_from(src_hbm, row_start):
            cp = pltpu.make_async_copy(src_hbm, stage, copy_sem)
            cp.start()
            cp.wait()
            acc = jnp.dot(stage[...], w_ref[...],
                          preferred_element_type=jnp.float32)
            out_ref[pl.ds(row_start, HALF), :] = jnp.maximum(acc, 0.0)

        for h in range(N_DEV - 1):
            send_slot = h % 2
            recv_slot = (h + 1) % 2
            cw_src = x_ref.at[pl.ds(0, HALF), :] if h == 0 else comm_cw.at[send_slot]
            ccw_src = x_ref.at[pl.ds(HALF, HALF), :] if h == 0 else comm_ccw.at[send_slot]
            cw = pltpu.make_async_remote_copy(
                src_ref=cw_src,
                dst_ref=comm_cw.at[recv_slot],
                send_sem=cw_send.at[send_slot],
                recv_sem=cw_recv.at[recv_slot],
                device_id=(right,),
                device_id_type=pl.DeviceIdType.MESH,
            )
            ccw = pltpu.make_async_remote_copy(
                src_ref=ccw_src,
                dst_ref=comm_ccw.at[recv_slot],
                send_sem=ccw_send.at[send_slot],
                recv_sem=ccw_recv.at[recv_slot],
                device_id=(left,),
                device_id_type=pl.DeviceIdType.MESH,
            )
            cw.start()
            ccw.start()

            if h == 0:
                gemm_from(x_ref.at[pl.ds(0, HALF), :], my * m_per)
                gemm_from(x_ref.at[pl.ds(HALF, HALF), :], my * m_per + HALF)

            cw.wait()
            ccw.wait()

            origin_cw = lax.rem(my - 1 - h + 2 * N_DEV, N_DEV)
            origin_ccw = lax.rem(my + 1 + h, N_DEV)
            gemm_from(comm_cw.at[recv_slot], origin_cw * m_per)
            gemm_from(comm_ccw.at[recv_slot], origin_ccw * m_per + HALF)

    return pl.pallas_call(
        body,
        out_shape=jax.ShapeDtypeStruct((N_DEV * m_per, n_per), jnp.float32),
        in_specs=[
            pl.BlockSpec(memory_space=pltpu.ANY),
            pl.BlockSpec(memory_space=pltpu.VMEM),
        ],
        out_specs=pl.BlockSpec(memory_space=pltpu.VMEM),
        scratch_shapes=[
            pltpu.ANY((2, HALF, k), jnp.float32),
            pltpu.ANY((2, HALF, k), jnp.float32),
            pltpu.VMEM((HALF, k), jnp.float32),
            pltpu.SemaphoreType.DMA((2,)),
            pltpu.SemaphoreType.DMA((2,)),
            pltpu.SemaphoreType.DMA((2,)),
            pltpu.SemaphoreType.DMA((2,)),
            pltpu.SemaphoreType.DMA,
        ],
        compiler_params=pltpu.CompilerParams(collective_id=0),
    )(x, w_mat)


# baseline (device time: 1455470 ns/iter reference)
import jax
import jax.numpy as jnp
from jax import lax
from jax.experimental import pallas as pl
from jax.experimental.pallas import tpu as pltpu

N_DEV = 8
HALF = 512


def kernel(x, w_mat):
    m_per, k = x.shape
    _, n_per = w_mat.shape
    assert m_per == 2 * HALF

    def body(x_ref, w_ref, out_ref, comm_cw, comm_ccw, stage,
             cw_send, cw_recv, ccw_send, ccw_recv, copy_sem):
        my = lax.axis_index("i")
        left = lax.rem(my - 1 + N_DEV, N_DEV)
        right = lax.rem(my + 1, N_DEV)

        barrier_sem = pltpu.get_barrier_semaphore()
        for nbr in (left, right):
            pl.semaphore_signal(
                barrier_sem, inc=1,
                device_id=(nbr,), device_id_type=pl.DeviceIdType.MESH,
            )
        pl.semaphore_wait(barrier_sem, 2)

        def gemm_from(src_hbm, row_start):
            cp = pltpu.make_async_copy(src_hbm, stage, copy_sem)
            cp.start()
            cp.wait()
            acc = jnp.dot(stage[...], w_ref[...],
                          preferred_element_type=jnp.float32)
            out_ref[pl.ds(row_start, HALF), :] = jnp.maximum(acc, 0.0)

        for h in range(N_DEV - 1):
            send_slot = h % 2
            recv_slot = (h + 1) % 2
            cw_src = x_ref.at[pl.ds(0, HALF), :] if h == 0 else comm_cw.at[send_slot]
            ccw_src = x_ref.at[pl.ds(HALF, HALF), :] if h == 0 else comm_ccw.at[send_slot]
            cw = pltpu.make_async_remote_copy(
                src_ref=cw_src,
                dst_ref=comm_cw.at[recv_slot],
                send_sem=cw_send.at[send_slot],
                recv_sem=cw_recv.at[recv_slot],
                device_id=(right,),
                device_id_type=pl.DeviceIdType.MESH,
            )
            ccw = pltpu.make_async_remote_copy(
                src_ref=ccw_src,
                dst_ref=comm_ccw.at[recv_slot],
                send_sem=ccw_send.at[send_slot],
                recv_sem=ccw_recv.at[recv_slot],
                device_id=(left,),
                device_id_type=pl.DeviceIdType.MESH,
            )
            cw.start()
            ccw.start()

            if h == 0:
                gemm_from(x_ref.at[pl.ds(0, HALF), :], my * m_per)
                gemm_from(x_ref.at[pl.ds(HALF, HALF), :], my * m_per + HALF)

            cw.wait()
            ccw.wait()

            origin_cw = lax.rem(my - 1 - h + 2 * N_DEV, N_DEV)
            origin_ccw = lax.rem(my + 1 + h, N_DEV)
            gemm_from(comm_cw.at[recv_slot], origin_cw * m_per)
            gemm_from(comm_ccw.at[recv_slot], origin_ccw * m_per + HALF)

    out, _, _ = pl.pallas_call(
        body,
        out_shape=(
            jax.ShapeDtypeStruct((N_DEV * m_per, n_per), jnp.float32),
            jax.ShapeDtypeStruct((2, HALF, k), jnp.float32),
            jax.ShapeDtypeStruct((2, HALF, k), jnp.float32),
        ),
        in_specs=[
            pl.BlockSpec(memory_space=pltpu.HBM),
            pl.BlockSpec(memory_space=pltpu.VMEM),
        ],
        out_specs=(
            pl.BlockSpec(memory_space=pltpu.VMEM),
            pl.BlockSpec(memory_space=pltpu.HBM),
            pl.BlockSpec(memory_space=pltpu.HBM),
        ),
        scratch_shapes=[
            pltpu.VMEM((HALF, k), jnp.float32),
            pltpu.SemaphoreType.DMA((2,)),
            pltpu.SemaphoreType.DMA((2,)),
            pltpu.SemaphoreType.DMA((2,)),
            pltpu.SemaphoreType.DMA((2,)),
            pltpu.SemaphoreType.DMA,
        ],
        compiler_params=pltpu.CompilerParams(
            collective_id=0,
            vmem_limit_bytes=64 * 1024 * 1024,
        ),
    )(x, w_mat)
    return out


# device time: 914792 ns/iter; 1.5910x vs baseline; 1.5910x over previous
import jax
import jax.numpy as jnp
from jax import lax
from jax.experimental import pallas as pl
from jax.experimental.pallas import tpu as pltpu

N_DEV = 8
N_STEPS = 7
M_PER = 1024
LENS = (344, 336, 344)
OFFS = (0, 344, 680)
BITPAT = ((1, 2, 4), (4, 1, 2), (2, 4, 1))
P_END = (4, 2, 1)


def _bit(t, h):
    e, o1, o3 = BITPAT[t]
    return jnp.where(h % 2 == 0, e, jnp.where(h % 4 == 1, o1, o3))


def kernel(x, w_mat):
    m_per, k = x.shape
    _, n_per = w_mat.shape
    assert m_per == M_PER

    def body(x_ref, w_ref, out_ref, c0, c1, c2, stage, send_sems, recv_sems,
             copy_sem):
        comm = (c0, c1, c2)
        my = lax.axis_index("i")

        s = lax.rem(my, 4)
        xb = jnp.where((s == 1) | (s == 2), 1, 0)
        yb = jnp.where(s >= 2, 1, 0)
        my_label = 4 * xb + 2 * yb + my // 4

        def pos_of_label(lab):
            lx = (lab // 4) % 2
            ly = (lab // 2) % 2
            lz = lab % 2
            return 4 * lz + 2 * ly + (lx ^ ly)

        def partner_pos(bit):
            return pos_of_label(my_label ^ bit)

        barrier_sem = pltpu.get_barrier_semaphore()
        for bit in (1, 2, 4):
            pl.semaphore_signal(
                barrier_sem, inc=1,
                device_id=(partner_pos(bit),),
                device_id_type=pl.DeviceIdType.MESH,
            )
        pl.semaphore_wait(barrier_sem, 3)

        def desc(t, src, dst_slot, h_sem, bit):
            return pltpu.make_async_remote_copy(
                src_ref=src,
                dst_ref=comm[t].at[dst_slot],
                send_sem=send_sems.at[t, h_sem],
                recv_sem=recv_sems.at[t, h_sem],
                device_id=(partner_pos(bit),),
                device_id_type=pl.DeviceIdType.MESH,
            )

        def gemm_from(src_hbm, t, row_start):
            ln = LENS[t]
            cp = pltpu.make_async_copy(
                src_hbm, stage.at[pl.ds(0, ln), :], copy_sem)
            cp.start()
            cp.wait()
            acc = jnp.dot(stage[pl.ds(0, ln), :], w_ref[...],
                          preferred_element_type=jnp.float32)
            out_ref[pl.ds(row_start, ln), :] = jnp.maximum(acc, 0.0)

        def x_slice(t):
            return x_ref.at[pl.ds(OFFS[t], LENS[t]), :]

        for t in range(3):
            desc(t, x_slice(t), 1, 0, BITPAT[t][0]).start()
        for t in range(3):
            gemm_from(x_slice(t), t, my * M_PER + OFFS[t])
        for t in range(3):
            desc(t, x_slice(t), 1, 0, BITPAT[t][0]).wait()
        for t in range(3):
            desc(t, comm[t].at[1], 0, 1, BITPAT[t][1]).start()
        for t in range(3):
            o_pos = pos_of_label(my_label ^ BITPAT[t][0])
            gemm_from(comm[t].at[1], t, o_pos * M_PER + OFFS[t])

        def step(h, px):
            src_slot = lax.rem(h, 2)
            dst_slot = lax.rem(h + 1, 2)
            new_px = []
            for t in range(3):
                b = _bit(t, h)
                desc(t, comm[t].at[src_slot], dst_slot, h, b).wait()
                new_px.append(px[t] ^ b)
            for t in range(3):
                desc(t, comm[t].at[dst_slot], src_slot, h + 1,
                     _bit(t, h + 1)).start()
            for t in range(3):
                o_pos = pos_of_label(my_label ^ new_px[t])
                gemm_from(comm[t].at[dst_slot], t, o_pos * M_PER + OFFS[t])
            return tuple(new_px)

        px0 = tuple(jnp.int32(BITPAT[t][0]) for t in range(3))
        lax.fori_loop(1, 6, step, px0, unroll=False)

        for t in range(3):
            desc(t, comm[t].at[0], 1, 6, BITPAT[t][0]).wait()
        for t in range(3):
            o_pos = pos_of_label(my_label ^ P_END[t])
            gemm_from(comm[t].at[1], t, o_pos * M_PER + OFFS[t])

    max_len = max(LENS)
    out, _, _, _ = pl.pallas_call(
        body,
        out_shape=(
            jax.ShapeDtypeStruct((N_DEV * m_per, n_per), jnp.float32),
            jax.ShapeDtypeStruct((2, LENS[0], k), jnp.float32),
            jax.ShapeDtypeStruct((2, LENS[1], k), jnp.float32),
            jax.ShapeDtypeStruct((2, LENS[2], k), jnp.float32),
        ),
        in_specs=[
            pl.BlockSpec(memory_space=pltpu.HBM),
            pl.BlockSpec(memory_space=pltpu.VMEM),
        ],
        out_specs=(
            pl.BlockSpec(memory_space=pltpu.VMEM),
            pl.BlockSpec(memory_space=pltpu.HBM),
            pl.BlockSpec(memory_space=pltpu.HBM),
            pl.BlockSpec(memory_space=pltpu.HBM),
        ),
        scratch_shapes=[
            pltpu.VMEM((max_len, k), jnp.float32),
            pltpu.SemaphoreType.DMA((3, N_STEPS)),
            pltpu.SemaphoreType.DMA((3, N_STEPS)),
            pltpu.SemaphoreType.DMA,
        ],
        compiler_params=pltpu.CompilerParams(
            collective_id=0,
            vmem_limit_bytes=64 * 1024 * 1024,
        ),
    )(x, w_mat)
    return out
